# baseline (speedup 1.0000x reference)
import functools

import jax
import jax.numpy as jnp
from jax import lax
from jax.experimental import pallas as pl
from jax.experimental.pallas import tpu as pltpu
from jax.experimental.pallas import tpu_sc as plsc

_NBINS = 128
_OUT = 16
_N = 1048576
_NC = 2
_NS = 16
_NW = _NC * _NS
_PTS_PER_W = _N // _NW
_CHUNK = 2048
_NCHUNK = _PTS_PER_W // _CHUNK
_GID = 128
_NG = _CHUNK // _GID
_LANES = 16


def _body(inp_hbm, grid_hbm, out_hbm, in_v, idx_v, rows_v, sem):
    wid = lax.axis_index("s") * _NC + lax.axis_index("c")
    lane = lax.iota(jnp.int32, _LANES)

    @pl.loop(0, _NCHUNK)
    def _chunk(ci):
        pbase = (wid * _NCHUNK + ci) * _CHUNK
        pltpu.sync_copy(inp_hbm.at[pl.ds(pbase * 3, _CHUNK * 3)], in_v)

        @pl.loop(0, _CHUNK // _LANES)
        def _idx(g):
            pos = lane * 3 + g * (_LANES * 3)
            x = plsc.load_gather(in_v, [pos])
            y = plsc.load_gather(in_v, [pos + 1])
            z = plsc.load_gather(in_v, [pos + 2])
            fmax = jnp.float32(_NBINS - 1)
            fzero = jnp.float32(0.0)
            ix = jnp.minimum(jnp.maximum(x * _NBINS, fzero), fmax).astype(jnp.int32)
            iy = jnp.minimum(jnp.maximum(y * _NBINS, fzero), fmax).astype(jnp.int32)
            iz = jnp.minimum(jnp.maximum(z * _NBINS, fzero), fmax).astype(jnp.int32)
            idx_v[pl.ds(g * _LANES, _LANES)] = (ix * _NBINS + iy) * _NBINS + iz

        copies = []
        for j in range(_NG):
            copies.append(
                pltpu.async_copy(
                    grid_hbm.at[idx_v.at[pl.ds(j * _GID, _GID)]],
                    rows_v.at[pl.ds(j * _GID, _GID)],
                    sem,
                )
            )
        for c in copies:
            c.wait()

        @pl.loop(0, _CHUNK)
        def _scale(i):
            rows_v[i] = rows_v[i] * jnp.float32(5.0)

        pltpu.sync_copy(rows_v, out_hbm.at[pl.ds(pbase, _CHUNK)])


@jax.jit
def _grid_gather(inputs_flat, grid_flat):
    mesh = plsc.VectorSubcoreMesh(core_axis_name="c", subcore_axis_name="s")
    return pl.kernel(
        _body,
        out_type=jax.ShapeDtypeStruct((_N, _OUT), jnp.float32),
        mesh=mesh,
        scratch_types=[
            pltpu.VMEM((_CHUNK * 3,), jnp.float32),
            pltpu.VMEM((_CHUNK,), jnp.int32),
            pltpu.VMEM((_CHUNK, _OUT), jnp.float32),
            pltpu.SemaphoreType.DMA,
        ],
        compiler_params=pltpu.CompilerParams(
            needs_layout_passes=False, use_tc_tiling_on_sc=False
        ),
    )(inputs_flat, grid_flat)


def kernel(inputs, grid):
    return _grid_gather(inputs.reshape(-1), grid.reshape(-1, _OUT))


# speedup vs baseline: 1.7090x; 1.7090x over previous
import jax
import jax.numpy as jnp
from jax import lax
from jax.experimental import pallas as pl
from jax.experimental.pallas import tpu as pltpu
from jax.experimental.pallas import tpu_sc as plsc

_NBINS = 128
_OUT = 16
_N = 1048576
_NFACE = _NBINS * _NBINS
_NCELL = _NFACE * _NBINS

_NW = 32
_PTS_PER_W = _N // _NW
_CHUNK = 1024
_NCH = _PTS_PER_W // _CHUNK
_GID = 128
_NG = _CHUNK // _GID
_IDXROWS_W = _PTS_PER_W // _GID


def _idx_body(x_ref, y_ref, z_ref, o_ref):
    fmax = jnp.float32(_NBINS - 1)
    fzero = jnp.float32(0.0)
    ix = jnp.minimum(jnp.maximum(x_ref[...] * _NBINS, fzero), fmax).astype(jnp.int32)
    iy = jnp.minimum(jnp.maximum(y_ref[...] * _NBINS, fzero), fmax).astype(jnp.int32)
    iz = jnp.minimum(jnp.maximum(z_ref[...] * _NBINS, fzero), fmax).astype(jnp.int32)
    o_ref[...] = (ix * _NBINS + iy) * _NBINS + iz


def _idx_tc(xs, ys, zs):
    nrow = _N // 128
    br = 512
    spec = pl.BlockSpec((br, 128), lambda i: (i, 0))
    return pl.pallas_call(
        _idx_body,
        grid=(nrow // br,),
        in_specs=[spec, spec, spec],
        out_specs=pl.BlockSpec((br, 128), lambda i: (i, 0)),
        out_shape=jax.ShapeDtypeStruct((nrow, 128), jnp.int32),
    )(xs, ys, zs)


def _tab_body(g_ref, o_ref):
    o_ref[...] = jnp.swapaxes(g_ref[...] * jnp.float32(5.0), 1, 2)


def _tab_tc(gv):
    bf = 64
    return pl.pallas_call(
        _tab_body,
        grid=(_NFACE // bf,),
        in_specs=[pl.BlockSpec((bf, _OUT, _NBINS), lambda i: (i, 0, 0))],
        out_specs=pl.BlockSpec((bf, _NBINS, _OUT), lambda i: (i, 0, 0)),
        out_shape=jax.ShapeDtypeStruct((_NFACE, _NBINS, _OUT), jnp.float32),
    )(gv)


def _gather_body(idx_hbm, tab_hbm, out_hbm, idx_v, rows0, rows1, sg0, sg1, so0, so1):
    wid = lax.axis_index("s") * 2 + lax.axis_index("c")
    row0 = wid * _IDXROWS_W
    pltpu.sync_copy(idx_hbm.at[pl.ds(row0, _IDXROWS_W)], idx_v)

    rows = (rows0, rows1)
    sg = (sg0, sg1)
    so = (so0, so1)

    def fire(c):
        buf = rows[c % 2]
        descs = []
        for g in range(_NG):
            descs.append(
                pltpu.async_copy(
                    tab_hbm.at[idx_v.at[c * _NG + g]],
                    buf.at[pl.ds(g * _GID, _GID)],
                    sg[c % 2],
                )
            )
        return descs

    gd = [None, None]
    outd = [None, None]
    gd[0] = fire(0)
    for c in range(_NCH):
        if c + 1 < _NCH:
            if c >= 1:
                outd[(c + 1) % 2].wait()
            gd[(c + 1) % 2] = fire(c + 1)
        for d in gd[c % 2]:
            d.wait()
        outd[c % 2] = pltpu.async_copy(
            rows[c % 2],
            out_hbm.at[pl.ds(wid * _PTS_PER_W + c * _CHUNK, _CHUNK)],
            so[c % 2],
        )
    outd[0].wait()
    outd[1].wait()


@jax.jit
def _run(inputs, grid):
    xs = inputs[:, 0].reshape(_N // 128, 128)
    ys = inputs[:, 1].reshape(_N // 128, 128)
    zs = inputs[:, 2].reshape(_N // 128, 128)
    idx = _idx_tc(xs, ys, zs).reshape(_NW * _IDXROWS_W, _GID)

    gv = jnp.transpose(grid, (0, 1, 3, 2)).reshape(_NFACE, _OUT, _NBINS)
    tab = _tab_tc(gv).reshape(_NCELL, _OUT)

    mesh = plsc.VectorSubcoreMesh(core_axis_name="c", subcore_axis_name="s")
    return pl.kernel(
        _gather_body,
        out_type=jax.ShapeDtypeStruct((_N, _OUT), jnp.float32),
        mesh=mesh,
        scratch_types=[
            pltpu.VMEM((_IDXROWS_W, _GID), jnp.int32),
            pltpu.VMEM((_CHUNK, _OUT), jnp.float32),
            pltpu.VMEM((_CHUNK, _OUT), jnp.float32),
            pltpu.SemaphoreType.DMA,
            pltpu.SemaphoreType.DMA,
            pltpu.SemaphoreType.DMA,
            pltpu.SemaphoreType.DMA,
        ],
        compiler_params=pltpu.CompilerParams(
            needs_layout_passes=False, use_tc_tiling_on_sc=False
        ),
    )(idx, tab)


def kernel(inputs, grid):
    return _run(inputs, grid)


# speedup vs baseline: 2.8194x; 1.6497x over previous
import jax
import jax.numpy as jnp
from jax import lax
from jax.experimental import pallas as pl
from jax.experimental.pallas import tpu as pltpu
from jax.experimental.pallas import tpu_sc as plsc

_NBINS = 128
_OUT = 16
_N = 1048576
_NFACE = _NBINS * _NBINS
_NCELL = _NFACE * _NBINS

_NW = 32
_PTS_PER_W = _N // _NW
_CHUNK = 1024
_NCH = _PTS_PER_W // _CHUNK
_GID = 128
_NG = _CHUNK // _GID
_IDXROWS_W = _PTS_PER_W // _GID
_HALF = _N * 8


def _idx_body(x_ref, y_ref, z_ref, o_ref):
    fmax = jnp.float32(_NBINS - 1)
    fzero = jnp.float32(0.0)
    ix = jnp.minimum(jnp.maximum(x_ref[...] * _NBINS, fzero), fmax).astype(jnp.int32)
    iy = jnp.minimum(jnp.maximum(y_ref[...] * _NBINS, fzero), fmax).astype(jnp.int32)
    iz = jnp.minimum(jnp.maximum(z_ref[...] * _NBINS, fzero), fmax).astype(jnp.int32)
    o_ref[...] = (ix * _NBINS + iy) * _NBINS + iz


def _idx_tc(xs, ys, zs):
    nrow = _N // 128
    br = 512
    spec = pl.BlockSpec((br, 128), lambda i: (i, 0))
    return pl.pallas_call(
        _idx_body,
        grid=(nrow // br,),
        in_specs=[spec, spec, spec],
        out_specs=pl.BlockSpec((br, 128), lambda i: (i, 0)),
        out_shape=jax.ShapeDtypeStruct((nrow, 128), jnp.int32),
    )(xs, ys, zs)


_BF = 64


def _tab_body(g_ref, o_ref):
    t = jnp.swapaxes(g_ref[...] * jnp.float32(5.0), 1, 2)
    t4 = t.reshape(_BF, 16, 8, 16)
    o_ref[...] = jnp.concatenate([t4[:, :, i, :] for i in range(8)], axis=2)


def _tab_tc(gv):
    return pl.pallas_call(
        _tab_body,
        grid=(_NFACE // _BF,),
        in_specs=[pl.BlockSpec((_BF, _OUT, _NBINS), lambda i: (i, 0, 0))],
        out_specs=pl.BlockSpec((_BF, _OUT, _NBINS), lambda i: (i, 0, 0)),
        out_shape=jax.ShapeDtypeStruct((_NFACE, _OUT, _NBINS), jnp.float32),
    )(gv)


def _gather_body(idx_hbm, tab_hbm, out_hbm, idx_v, rows0, rows1, tch0, tch1,
                 sg0, sg1, so0, so1):
    wid = lax.axis_index("s") * 2 + lax.axis_index("c")
    row0 = wid * _IDXROWS_W
    pltpu.sync_copy(idx_hbm.at[pl.ds(row0, _IDXROWS_W)], idx_v)

    rows = (rows0, rows1)
    tch = (tch0, tch1)
    sg = (sg0, sg1)
    so = (so0, so1)
    lanes = lax.iota(jnp.int32, 16)
    cvec = (lanes // 8) * 8192 + (lanes % 8) * 128

    def fire(c):
        buf = rows[c % 2]
        descs = []
        for g in range(_NG):
            descs.append(
                pltpu.async_copy(
                    tab_hbm.at[idx_v.at[c * _NG + g]],
                    buf.at[pl.ds(g * _GID, _GID)],
                    sg[c % 2],
                )
            )
        return descs

    gd = [None, None]
    outd = [None, None]
    gd[0] = fire(0)
    for c in range(_NCH):
        if c + 1 < _NCH:
            gd[(c + 1) % 2] = fire(c + 1)
        for d in gd[c % 2]:
            d.wait()
        if c >= 2:
            for d in outd[c % 2]:
                d.wait()
        rbuf = rows[c % 2]
        tbuf = tch[c % 2]

        @pl.loop(0, _CHUNK, unroll=8)
        def _tp(p):
            g = p >> 7
            nl = p & 127
            v = rbuf[p]
            plsc.store_scatter(tbuf, [cvec + (g * 1024 + nl)], v)

        nb0 = wid * _IDXROWS_W + c * _NG
        outd[c % 2] = [
            pltpu.async_copy(
                tbuf.at[pl.ds(0, _CHUNK * 8)],
                out_hbm.at[pl.ds(nb0 * 1024, _CHUNK * 8)],
                so[c % 2],
            ),
            pltpu.async_copy(
                tbuf.at[pl.ds(_CHUNK * 8, _CHUNK * 8)],
                out_hbm.at[pl.ds(_HALF + nb0 * 1024, _CHUNK * 8)],
                so[c % 2],
            ),
        ]
    for c in (_NCH - 2, _NCH - 1):
        for d in outd[c % 2]:
            d.wait()


@jax.jit
def _run(inputs, grid):
    xs = inputs[:, 0].reshape(_N // 128, 128)
    ys = inputs[:, 1].reshape(_N // 128, 128)
    zs = inputs[:, 2].reshape(_N // 128, 128)
    idx = _idx_tc(xs, ys, zs).reshape(_NW * _IDXROWS_W, _GID)

    gv = jnp.transpose(grid, (0, 1, 3, 2)).reshape(_NFACE, _OUT, _NBINS)
    tab = _tab_tc(gv).reshape(_NCELL, _OUT)

    mesh = plsc.VectorSubcoreMesh(core_axis_name="c", subcore_axis_name="s")
    y = pl.kernel(
        _gather_body,
        out_type=jax.ShapeDtypeStruct((2 * _HALF,), jnp.float32),
        mesh=mesh,
        scratch_types=[
            pltpu.VMEM((_IDXROWS_W, _GID), jnp.int32),
            pltpu.VMEM((_CHUNK, _OUT), jnp.float32),
            pltpu.VMEM((_CHUNK, _OUT), jnp.float32),
            pltpu.VMEM((_CHUNK * _OUT,), jnp.float32),
            pltpu.VMEM((_CHUNK * _OUT,), jnp.float32),
            pltpu.SemaphoreType.DMA,
            pltpu.SemaphoreType.DMA,
            pltpu.SemaphoreType.DMA,
            pltpu.SemaphoreType.DMA,
        ],
        compiler_params=pltpu.CompilerParams(
            needs_layout_passes=False, use_tc_tiling_on_sc=False
        ),
    )(idx, tab)
    return y.reshape(2, _N // 128, 8, 128).transpose(1, 3, 0, 2).reshape(_N, _OUT)


def kernel(inputs, grid):
    return _run(inputs, grid)


# speedup vs baseline: 5.4631x; 1.9377x over previous
import jax
import jax.numpy as jnp
from jax import lax
from jax.experimental import pallas as pl
from jax.experimental.pallas import tpu as pltpu
from jax.experimental.pallas import tpu_sc as plsc

_NBINS = 128
_OUT = 16
_N = 1048576
_NFACE = _NBINS * _NBINS
_NCELL = _NFACE * _NBINS

_NW = 32
_PTS_PER_W = _N // _NW
_CHUNK = 1024
_NCH = _PTS_PER_W // _CHUNK
_GID = 128
_NG = _CHUNK // _GID
_IDXROWS_W = _PTS_PER_W // _GID
_HALF = _N * 8


def _idx_body(x_ref, y_ref, z_ref, o_ref):
    fmax = jnp.float32(_NBINS - 1)
    fzero = jnp.float32(0.0)
    ix = jnp.minimum(jnp.maximum(x_ref[...] * _NBINS, fzero), fmax).astype(jnp.int32)
    iy = jnp.minimum(jnp.maximum(y_ref[...] * _NBINS, fzero), fmax).astype(jnp.int32)
    iz = jnp.minimum(jnp.maximum(z_ref[...] * _NBINS, fzero), fmax).astype(jnp.int32)
    o_ref[...] = (ix * 16 + (iy >> 3)) * 1024 + iz * 8 + (iy & 7)


def _idx_tc(xs, ys, zs):
    nrow = _N // 128
    br = 512
    spec = pl.BlockSpec((br, 128), lambda i: (i, 0))
    return pl.pallas_call(
        _idx_body,
        grid=(nrow // br,),
        in_specs=[spec, spec, spec],
        out_specs=pl.BlockSpec((br, 128), lambda i: (i, 0)),
        out_shape=jax.ShapeDtypeStruct((nrow, 128), jnp.int32),
    )(xs, ys, zs)


_BF8 = 16
_NF8 = _NFACE // 8


def _tab_body(g_ref, o_ref):
    o_ref[...] = jnp.swapaxes(g_ref[...] * jnp.float32(5.0), 1, 2)


def _tab_tc(gv):
    return pl.pallas_call(
        _tab_body,
        grid=(_NF8 // _BF8,),
        in_specs=[pl.BlockSpec((_BF8, 128, 128), lambda i: (i, 0, 0))],
        out_specs=pl.BlockSpec((_BF8, 128, 128), lambda i: (i, 0, 0)),
        out_shape=jax.ShapeDtypeStruct((_NF8, 128, 128), jnp.float32),
    )(gv)


def _gather_body(idx_hbm, tab_hbm, out_hbm, idx_v, rows0, rows1, tch0, tch1,
                 sg0, sg1, so0, so1):
    wid = lax.axis_index("s") * 2 + lax.axis_index("c")
    row0 = wid * _IDXROWS_W
    pltpu.sync_copy(idx_hbm.at[pl.ds(row0, _IDXROWS_W)], idx_v)

    rows = (rows0, rows1)
    tch = (tch0, tch1)
    sg = (sg0, sg1)
    so = (so0, so1)
    lanes = lax.iota(jnp.int32, 16)
    cvec = (lanes // 8) * 8192 + (lanes % 8) * 128

    def fire(c):
        buf = rows[c % 2]
        descs = []
        for g in range(_NG):
            descs.append(
                pltpu.async_copy(
                    tab_hbm.at[idx_v.at[c * _NG + g]],
                    buf.at[pl.ds(g * _GID, _GID)],
                    sg[c % 2],
                )
            )
        return descs

    gd = [None, None]
    outd = [None, None]
    gd[0] = fire(0)
    for c in range(_NCH):
        if c + 1 < _NCH:
            gd[(c + 1) % 2] = fire(c + 1)
        for d in gd[c % 2]:
            d.wait()
        if c >= 2:
            for d in outd[c % 2]:
                d.wait()
        rbuf = rows[c % 2]
        tbuf = tch[c % 2]

        @pl.loop(0, _CHUNK, unroll=8)
        def _tp(p):
            g = p >> 7
            nl = p & 127
            v = rbuf[p]
            plsc.store_scatter(tbuf, [cvec + (g * 1024 + nl)], v)

        nb0 = wid * _IDXROWS_W + c * _NG
        outd[c % 2] = [
            pltpu.async_copy(
                tbuf.at[pl.ds(0, _CHUNK * 8)],
                out_hbm.at[pl.ds(nb0 * 1024, _CHUNK * 8)],
                so[c % 2],
            ),
            pltpu.async_copy(
                tbuf.at[pl.ds(_CHUNK * 8, _CHUNK * 8)],
                out_hbm.at[pl.ds(_HALF + nb0 * 1024, _CHUNK * 8)],
                so[c % 2],
            ),
        ]
    for c in (_NCH - 2, _NCH - 1):
        for d in outd[c % 2]:
            d.wait()


@jax.jit
def _run(inputs, grid):
    xs = inputs[:, 0].reshape(_N // 128, 128)
    ys = inputs[:, 1].reshape(_N // 128, 128)
    zs = inputs[:, 2].reshape(_N // 128, 128)
    idx = _idx_tc(xs, ys, zs).reshape(_NW * _IDXROWS_W, _GID)

    gv = jnp.transpose(grid, (0, 1, 3, 2)).reshape(_NF8, 128, 128)
    tab = _tab_tc(gv).reshape(_NCELL, _OUT)

    mesh = plsc.VectorSubcoreMesh(core_axis_name="c", subcore_axis_name="s")
    y = pl.kernel(
        _gather_body,
        out_type=jax.ShapeDtypeStruct((2 * _HALF,), jnp.float32),
        mesh=mesh,
        scratch_types=[
            pltpu.VMEM((_IDXROWS_W, _GID), jnp.int32),
            pltpu.VMEM((_CHUNK, _OUT), jnp.float32),
            pltpu.VMEM((_CHUNK, _OUT), jnp.float32),
            pltpu.VMEM((_CHUNK * _OUT,), jnp.float32),
            pltpu.VMEM((_CHUNK * _OUT,), jnp.float32),
            pltpu.SemaphoreType.DMA,
            pltpu.SemaphoreType.DMA,
            pltpu.SemaphoreType.DMA,
            pltpu.SemaphoreType.DMA,
        ],
        compiler_params=pltpu.CompilerParams(
            needs_layout_passes=False, use_tc_tiling_on_sc=False
        ),
    )(idx, tab)
    return y.reshape(2, _N // 128, 8, 128).transpose(1, 3, 0, 2).reshape(_N, _OUT)


def kernel(inputs, grid):
    return _run(inputs, grid)
